# baseline (speedup 1.0000x reference)
import functools

import jax
import jax.numpy as jnp
from jax import lax
from jax.experimental import pallas as pl
from jax.experimental.pallas import tpu as pltpu
from jax.experimental.pallas import tpu_sc as plsc

N = 10000
DEG = 16
HID = 128
TDIM = 32
EDIM = 16
OUT = 128
T_NORM = (1.0 / (TDIM // 2)) ** 0.5
EPS = 1e-5

NW = 32
N_PAD = 10240
NPW = N_PAD // NW
CN = 8
BN = 256
GRID = N_PAD // BN


def _ln(h, g, b):
    m = jnp.mean(h, axis=-1, keepdims=True)
    v = jnp.mean((h - m) ** 2, axis=-1, keepdims=True)
    return (h - m) * jax.lax.rsqrt(v + EPS) * g + b


def _pre_body(st_ref, et_ref, x_ref, t_ref, r_ref, nb_ref, Wp_ref, bp_ref,
              R_ref, wt_ref, bt_ref, SWs_ref, SWc_ref, SWe_ref,
              g_ref, b_ref, Wv_ref, tbm_ref, idx_ref, xn_ref, v_ref):
    st = st_ref[0, 0]
    et = et_ref[0, 0]
    t = t_ref[...]
    mask = (t >= st) & (t < et)
    maskf = mask.astype(jnp.float32)
    anymask = jnp.max(maskf, axis=1, keepdims=True)
    tmax = jnp.maximum(st, jnp.max(jnp.where(mask, t, -jnp.inf), axis=1,
                                   keepdims=True))
    Rm = R_ref[...]
    t_rep = jnp.dot(t, Rm, preferred_element_type=jnp.float32)
    m_rep = jnp.dot(maskf, Rm, preferred_element_type=jnp.float32)
    hh = (tmax - t_rep) * wt_ref[...] + bt_ref[...]
    tb = (jnp.dot(m_rep * jnp.sin(hh), SWs_ref[...],
                  preferred_element_type=jnp.float32)
          + jnp.dot(m_rep * jnp.cos(hh), SWc_ref[...],
                    preferred_element_type=jnp.float32)) * T_NORM
    tb = tb + jnp.dot(m_rep * r_ref[...], SWe_ref[...],
                      preferred_element_type=jnp.float32)
    tbm_ref[...] = anymask * tb
    h0 = jnp.maximum(
        jnp.dot(x_ref[...], Wp_ref[...], preferred_element_type=jnp.float32)
        + bp_ref[...], 0.0)
    xn = _ln(h0, g_ref[...], b_ref[...])
    xn_ref[...] = xn
    v = jnp.dot(xn, Wv_ref[...], preferred_element_type=jnp.float32)
    row = (pl.program_id(0) * BN
           + lax.broadcasted_iota(jnp.int32, (BN, 1), 0))
    v_ref[...] = jnp.where(row < N, v, 0.0)
    idx_ref[...] = jnp.where(mask, nb_ref[...], N)


def _mlp_lnv_body(xn_ref, agg_ref, tbm_ref, g2_ref, b2_ref, W1a_ref, W1b_ref,
                  bl1_ref, W2_ref, bl2_ref, g1_ref, b1_ref, Wv_ref,
                  xn2_ref, v_ref):
    xn = xn_ref[...]
    h2 = agg_ref[...] + tbm_ref[...] + xn
    hn = _ln(h2, g2_ref[...], b2_ref[...])
    z = jnp.maximum(
        jnp.dot(xn, W1a_ref[...], preferred_element_type=jnp.float32)
        + jnp.dot(hn, W1b_ref[...], preferred_element_type=jnp.float32)
        + bl1_ref[...], 0.0)
    z = jnp.dot(z, W2_ref[...], preferred_element_type=jnp.float32) + bl2_ref[...]
    h = z + h2
    xn2 = _ln(h, g1_ref[...], b1_ref[...])
    xn2_ref[...] = xn2
    v = jnp.dot(xn2, Wv_ref[...], preferred_element_type=jnp.float32)
    row = (pl.program_id(0) * BN
           + lax.broadcasted_iota(jnp.int32, (BN, 1), 0))
    v_ref[...] = jnp.where(row < N, v, 0.0)


NCH = NPW // CN
NBUF = 4


@functools.cache
def _make_gather_sum():
    mesh = plsc.VectorSubcoreMesh(core_axis_name="c", subcore_axis_name="s")

    @functools.partial(
        pl.kernel,
        out_type=jax.ShapeDtypeStruct((N_PAD, HID), jnp.float32),
        mesh=mesh,
        scratch_types=[
            pltpu.VMEM((NCH, CN * DEG), jnp.int32),
            pltpu.VMEM((CN * DEG,), jnp.int32),
            pltpu.VMEM((CN * DEG,), jnp.int32),
            pltpu.VMEM((CN * DEG,), jnp.int32),
            pltpu.VMEM((CN * DEG,), jnp.int32),
            pltpu.VMEM((CN, HID), jnp.float32),
            pltpu.VMEM((NBUF, CN * DEG, HID), jnp.float32),
            pltpu.VMEM_SHARED((16 * NBUF * CN, HID), jnp.float32),
            pltpu.SemaphoreType.DMA,
            pltpu.SemaphoreType.DMA,
            pltpu.SemaphoreType.DMA,
            pltpu.SemaphoreType.DMA,
            pltpu.SemaphoreType.DMA,
            pltpu.SemaphoreType.DMA,
            pltpu.SemaphoreType.DMA,
            pltpu.SemaphoreType.DMA,
        ],
    )
    def _gather_sum(v_hbm, idx_hbm, didx_hbm, out_hbm, idx_s, d0, d1, d2, d3,
                    zero_s, rows_s, acc_sh, g0, g1, g2, g3, o0, o1, o2, o3):
        didx_s = (d0, d1, d2, d3)
        gsem = (g0, g1, g2, g3)
        osem = (o0, o1, o2, o3)
        sid = lax.axis_index("s")
        wid = sid * 2 + lax.axis_index("c")
        base = wid * NPW
        pltpu.sync_copy(idx_hbm.at[pl.ds(wid * NCH, NCH)], idx_s)
        for b in range(NBUF):
            pltpu.sync_copy(
                didx_hbm.at[pl.ds((sid * NBUF + b) * (CN * DEG), CN * DEG)],
                didx_s[b])
        for c in range(CN):
            for j in range(HID // 16):
                zero_s[c, pl.ds(j * 16, 16)] = jnp.zeros((16,), jnp.float32)

        def acc_rows(b):
            return acc_sh.at[pl.ds(sid * (NBUF * CN) + b * CN, CN)]

        def issue_gather(ci, b):
            return pltpu.async_copy(v_hbm.at[idx_s.at[ci]], rows_s.at[b],
                                    gsem[b])

        for b in range(NBUF):
            issue_gather(b, b)

        def group(g, carry):
            for b in range(NBUF):
                ci = g * NBUF + b
                node0 = base + ci * CN
                pltpu.make_async_copy(v_hbm.at[idx_s.at[ci]], rows_s.at[b],
                                      gsem[b]).wait()

                @pl.when(g > 0)
                def _wait_out():
                    pltpu.make_async_copy(
                        acc_rows(b), out_hbm.at[pl.ds(node0, CN)],
                        osem[b]).wait()

                pltpu.sync_copy(zero_s, acc_rows(b))
                pltpu.sync_copy(rows_s.at[b], acc_sh.at[didx_s[b]],
                                add=True)
                plsc.subcore_barrier()
                pltpu.async_copy(acc_rows(b), out_hbm.at[pl.ds(node0, CN)],
                                 osem[b])
                nc = ci + NBUF

                @pl.when(nc < NCH)
                def _next():
                    issue_gather(nc, b)

            return carry

        lax.fori_loop(0, NCH // NBUF, group, 0)
        for b in range(NBUF):
            node0 = base + (NCH - NBUF + b) * CN
            pltpu.make_async_copy(acc_rows(b), out_hbm.at[pl.ds(node0, CN)],
                                  osem[b]).wait()

    return _gather_sum


def _mlp_final_body(xn_ref, agg_ref, tbm_ref, g_ref, b_ref, W1a_ref, W1b_ref,
                    bl1_ref, W2_ref, bl2_ref, Wo_ref, bo_ref, o_ref):
    xn = xn_ref[...]
    h2 = agg_ref[...] + tbm_ref[...] + xn
    hn = _ln(h2, g_ref[...], b_ref[...])
    z = jnp.maximum(
        jnp.dot(xn, W1a_ref[...], preferred_element_type=jnp.float32)
        + jnp.dot(hn, W1b_ref[...], preferred_element_type=jnp.float32)
        + bl1_ref[...], 0.0)
    z = jnp.dot(z, W2_ref[...], preferred_element_type=jnp.float32) + bl2_ref[...]
    hnxt = z + h2
    o_ref[...] = (jnp.dot(hnxt, Wo_ref[...], preferred_element_type=jnp.float32)
                  + bo_ref[...])


def _row_spec():
    return pl.BlockSpec((BN, HID), lambda i: (i, 0))


def _full_spec(shape):
    return pl.BlockSpec(shape, lambda i: tuple(0 for _ in shape))


def kernel(x, neighbors, times, rels, start_t, end_t, Wp, bp, ln1_g, ln1_b,
           Wkqv, Wt, bt, Wtime, Wedge, ln2_g, ln2_b, Wl1, bl1, Wl2, bl2,
           Wout, bout):
    f32 = jnp.float32
    st = jnp.asarray(start_t, f32).reshape(1, 1)
    et = jnp.asarray(end_t, f32).reshape(1, 1)

    pad = N_PAD - N
    x_p = jnp.pad(x, ((0, pad), (0, 0)))
    t_p = jnp.pad(times[:, :, 0], ((0, pad), (0, 0)), constant_values=-1.0)
    r_p = jnp.pad(rels.reshape(N, DEG * EDIM), ((0, pad), (0, 0)))
    nb_p = jnp.pad(neighbors.astype(jnp.int32), ((0, pad), (0, 0)))
    Wv = Wkqv[:, 2 * HID:]
    Wtv = Wtime[:, 2 * HID:]
    Wts, Wtc = Wtv[0::2], Wtv[1::2]
    We = Wedge[:, 2 * HID:]
    W1a, W1b = Wl1[:HID], Wl1[HID:]
    bp2 = bp.reshape(1, HID)
    LREP = DEG * (TDIM // 2)
    Rm = jnp.repeat(jnp.eye(DEG, dtype=f32), TDIM // 2, axis=1)
    wt_t = jnp.tile(Wt.reshape(1, TDIM // 2), (1, DEG))
    bt_t = jnp.tile(bt.reshape(1, TDIM // 2), (1, DEG))
    SWs = jnp.tile(Wts, (DEG, 1))
    SWc = jnp.tile(Wtc, (DEG, 1))
    SWe = jnp.tile(We, (DEG, 1))
    g1, b1 = ln1_g.reshape(1, HID), ln1_b.reshape(1, HID)
    g2, b2 = ln2_g.reshape(1, HID), ln2_b.reshape(1, HID)
    bl1r, bl2r = bl1.reshape(1, HID), bl2.reshape(1, HID)
    bor = bout.reshape(1, OUT)

    tbm, idx2d, xn, v = pl.pallas_call(
        _pre_body,
        grid=(GRID,),
        in_specs=[
            _full_spec((1, 1)), _full_spec((1, 1)),
            _row_spec(),
            pl.BlockSpec((BN, DEG), lambda i: (i, 0)),
            pl.BlockSpec((BN, DEG * EDIM), lambda i: (i, 0)),
            pl.BlockSpec((BN, DEG), lambda i: (i, 0)),
            _full_spec((HID, HID)), _full_spec((1, HID)),
            _full_spec((DEG, LREP)),
            _full_spec((1, LREP)), _full_spec((1, LREP)),
            _full_spec((LREP, HID)), _full_spec((LREP, HID)),
            _full_spec((LREP, HID)),
            _full_spec((1, HID)), _full_spec((1, HID)),
            _full_spec((HID, HID)),
        ],
        out_specs=[_row_spec(),
                   pl.BlockSpec((BN, DEG), lambda i: (i, 0)),
                   _row_spec(), _row_spec()],
        out_shape=[
            jax.ShapeDtypeStruct((N_PAD, HID), f32),
            jax.ShapeDtypeStruct((N_PAD, DEG), jnp.int32),
            jax.ShapeDtypeStruct((N_PAD, HID), f32),
            jax.ShapeDtypeStruct((N_PAD, HID), f32),
        ],
    )(st, et, x_p, t_p, r_p, nb_p, Wp, bp2, Rm, wt_t, bt_t, SWs, SWc, SWe,
      g1, b1, Wv)
    idx2d = idx2d.reshape(N_PAD // CN, CN * DEG)

    didx = (jnp.arange(16, dtype=jnp.int32)[:, None, None] * (NBUF * CN)
            + jnp.arange(NBUF, dtype=jnp.int32)[None, :, None] * CN
            + jnp.repeat(jnp.arange(CN, dtype=jnp.int32), DEG)[None, None, :]
            ).reshape(16 * NBUF * CN * DEG)

    mlp_lnv = pl.pallas_call(
        _mlp_lnv_body,
        grid=(GRID,),
        in_specs=[_row_spec(), _row_spec(), _row_spec(),
                  _full_spec((1, HID)), _full_spec((1, HID)),
                  _full_spec((HID, HID)), _full_spec((HID, HID)),
                  _full_spec((1, HID)), _full_spec((HID, HID)),
                  _full_spec((1, HID)),
                  _full_spec((1, HID)), _full_spec((1, HID)),
                  _full_spec((HID, HID))],
        out_specs=[_row_spec(), _row_spec()],
        out_shape=[jax.ShapeDtypeStruct((N_PAD, HID), f32),
                   jax.ShapeDtypeStruct((N_PAD, HID), f32)],
    )

    mlp_final = pl.pallas_call(
        _mlp_final_body,
        grid=(GRID,),
        in_specs=[_row_spec(), _row_spec(), _row_spec(),
                  _full_spec((1, HID)), _full_spec((1, HID)),
                  _full_spec((HID, HID)), _full_spec((HID, HID)),
                  _full_spec((1, HID)), _full_spec((HID, HID)),
                  _full_spec((1, HID)), _full_spec((HID, OUT)),
                  _full_spec((1, OUT))],
        out_specs=pl.BlockSpec((BN, OUT), lambda i: (i, 0)),
        out_shape=jax.ShapeDtypeStruct((N_PAD, OUT), f32),
    )

    agg = _make_gather_sum()(v, idx2d, didx)
    xn, v = mlp_lnv(xn, agg, tbm, g2, b2, W1a, W1b, bl1r, Wl2, bl2r,
                    g1, b1, Wv)
    agg = _make_gather_sum()(v, idx2d, didx)
    out = mlp_final(xn, agg, tbm, g2, b2, W1a, W1b, bl1r, Wl2, bl2r, Wout, bor)
    return out[:N]


# speedup vs baseline: 1.0014x; 1.0014x over previous
import functools

import jax
import jax.numpy as jnp
from jax import lax
from jax.experimental import pallas as pl
from jax.experimental.pallas import tpu as pltpu
from jax.experimental.pallas import tpu_sc as plsc

N = 10000
DEG = 16
HID = 128
TDIM = 32
EDIM = 16
OUT = 128
T_NORM = (1.0 / (TDIM // 2)) ** 0.5
EPS = 1e-5

NW = 32
N_PAD = 10240
NPW = N_PAD // NW
CN = 8
BN = 256
GRID = N_PAD // BN


def _ln(h, g, b):
    m = jnp.mean(h, axis=-1, keepdims=True)
    v = jnp.mean((h - m) ** 2, axis=-1, keepdims=True)
    return (h - m) * jax.lax.rsqrt(v + EPS) * g + b


def _pre_body(st_ref, et_ref, x_ref, t_ref, r_ref, nb_ref, Wp_ref, bp_ref,
              R_ref, wt_ref, bt_ref, SWs_ref, SWc_ref, SWe_ref,
              g_ref, b_ref, Wv_ref, tbm_ref, idx_ref, xn_ref, v_ref):
    st = st_ref[0, 0]
    et = et_ref[0, 0]
    t = t_ref[...]
    mask = (t >= st) & (t < et)
    maskf = mask.astype(jnp.float32)
    anymask = jnp.max(maskf, axis=1, keepdims=True)
    tmax = jnp.maximum(st, jnp.max(jnp.where(mask, t, -jnp.inf), axis=1,
                                   keepdims=True))
    Rm = R_ref[...]
    t_rep = jnp.dot(t, Rm, preferred_element_type=jnp.float32)
    m_rep = jnp.dot(maskf, Rm, preferred_element_type=jnp.float32)
    hh = (tmax - t_rep) * wt_ref[...] + bt_ref[...]
    tb = (jnp.dot(m_rep * jnp.sin(hh), SWs_ref[...],
                  preferred_element_type=jnp.float32)
          + jnp.dot(m_rep * jnp.cos(hh), SWc_ref[...],
                    preferred_element_type=jnp.float32)) * T_NORM
    tb = tb + jnp.dot(m_rep * r_ref[...], SWe_ref[...],
                      preferred_element_type=jnp.float32)
    tbm_ref[...] = anymask * tb
    h0 = jnp.maximum(
        jnp.dot(x_ref[...], Wp_ref[...], preferred_element_type=jnp.float32)
        + bp_ref[...], 0.0)
    xn = _ln(h0, g_ref[...], b_ref[...])
    xn_ref[...] = xn
    v = jnp.dot(xn, Wv_ref[...], preferred_element_type=jnp.float32)
    row = (pl.program_id(0) * BN
           + lax.broadcasted_iota(jnp.int32, (BN, 1), 0))
    v_ref[...] = jnp.where(row < N, v, 0.0)
    idx_ref[...] = jnp.where(mask, nb_ref[...], N)


def _mlp_lnv_body(xn_ref, agg_ref, tbm_ref, g2_ref, b2_ref, W1a_ref, W1b_ref,
                  bl1_ref, W2_ref, bl2_ref, g1_ref, b1_ref, Wv_ref,
                  xn2_ref, v_ref):
    xn = xn_ref[...]
    h2 = agg_ref[...] + tbm_ref[...] + xn
    hn = _ln(h2, g2_ref[...], b2_ref[...])
    z = jnp.maximum(
        jnp.dot(xn, W1a_ref[...], preferred_element_type=jnp.float32)
        + jnp.dot(hn, W1b_ref[...], preferred_element_type=jnp.float32)
        + bl1_ref[...], 0.0)
    z = jnp.dot(z, W2_ref[...], preferred_element_type=jnp.float32) + bl2_ref[...]
    h = z + h2
    xn2 = _ln(h, g1_ref[...], b1_ref[...])
    xn2_ref[...] = xn2
    v = jnp.dot(xn2, Wv_ref[...], preferred_element_type=jnp.float32)
    row = (pl.program_id(0) * BN
           + lax.broadcasted_iota(jnp.int32, (BN, 1), 0))
    v_ref[...] = jnp.where(row < N, v, 0.0)


NCH = NPW // CN
NBUF = 4


@functools.cache
def _make_gather_sum():
    mesh = plsc.VectorSubcoreMesh(core_axis_name="c", subcore_axis_name="s")

    @functools.partial(
        pl.kernel,
        out_type=jax.ShapeDtypeStruct((N_PAD, HID), jnp.float32),
        mesh=mesh,
        scratch_types=[
            pltpu.VMEM((NCH, CN * DEG), jnp.int32),
            pltpu.VMEM((CN * DEG,), jnp.int32),
            pltpu.VMEM((CN * DEG,), jnp.int32),
            pltpu.VMEM((CN * DEG,), jnp.int32),
            pltpu.VMEM((CN * DEG,), jnp.int32),
            pltpu.VMEM((CN, HID), jnp.float32),
            pltpu.VMEM((NBUF, CN * DEG, HID), jnp.float32),
            pltpu.VMEM_SHARED((16 * NBUF * CN, HID), jnp.float32),
            pltpu.SemaphoreType.DMA,
            pltpu.SemaphoreType.DMA,
            pltpu.SemaphoreType.DMA,
            pltpu.SemaphoreType.DMA,
            pltpu.SemaphoreType.DMA,
            pltpu.SemaphoreType.DMA,
            pltpu.SemaphoreType.DMA,
            pltpu.SemaphoreType.DMA,
        ],
    )
    def _gather_sum(v_hbm, idx_hbm, didx_hbm, out_hbm, idx_s, d0, d1, d2, d3,
                    zero_s, rows_s, acc_sh, g0, g1, g2, g3, o0, o1, o2, o3):
        didx_s = (d0, d1, d2, d3)
        gsem = (g0, g1, g2, g3)
        osem = (o0, o1, o2, o3)
        sid = lax.axis_index("s")
        wid = sid * 2 + lax.axis_index("c")
        base = wid * NPW
        pltpu.sync_copy(idx_hbm.at[pl.ds(wid * NCH, NCH)], idx_s)
        for b in range(NBUF):
            pltpu.sync_copy(
                didx_hbm.at[pl.ds((sid * NBUF + b) * (CN * DEG), CN * DEG)],
                didx_s[b])
        for c in range(CN):
            for j in range(HID // 16):
                zero_s[c, pl.ds(j * 16, 16)] = jnp.zeros((16,), jnp.float32)

        def acc_rows(b):
            return acc_sh.at[pl.ds(sid * (NBUF * CN) + b * CN, CN)]

        def issue_gather(ci, b):
            return pltpu.async_copy(v_hbm.at[idx_s.at[ci]], rows_s.at[b],
                                    gsem[b])

        for b in range(NBUF):
            issue_gather(b, b)

        def group(g, carry):
            for b in range(NBUF):
                ci = g * NBUF + b
                node0 = base + ci * CN
                pltpu.make_async_copy(v_hbm.at[idx_s.at[ci]], rows_s.at[b],
                                      gsem[b]).wait()

                @pl.when(g > 0)
                def _wait_out():
                    pltpu.make_async_copy(
                        acc_rows(b), out_hbm.at[pl.ds(node0, CN)],
                        osem[b]).wait()

                pltpu.sync_copy(zero_s, acc_rows(b))
                pltpu.sync_copy(rows_s.at[b], acc_sh.at[didx_s[b]],
                                add=True)
                nc = ci + NBUF

                @pl.when(nc < NCH)
                def _next():
                    issue_gather(nc, b)

            plsc.subcore_barrier()
            for b in range(NBUF):
                node0 = base + (g * NBUF + b) * CN
                pltpu.async_copy(acc_rows(b), out_hbm.at[pl.ds(node0, CN)],
                                 osem[b])
            return carry

        lax.fori_loop(0, NCH // NBUF, group, 0)
        for b in range(NBUF):
            node0 = base + (NCH - NBUF + b) * CN
            pltpu.make_async_copy(acc_rows(b), out_hbm.at[pl.ds(node0, CN)],
                                  osem[b]).wait()

    return _gather_sum


def _mlp_final_body(xn_ref, agg_ref, tbm_ref, g_ref, b_ref, W1a_ref, W1b_ref,
                    bl1_ref, W2_ref, bl2_ref, Wo_ref, bo_ref, o_ref):
    xn = xn_ref[...]
    h2 = agg_ref[...] + tbm_ref[...] + xn
    hn = _ln(h2, g_ref[...], b_ref[...])
    z = jnp.maximum(
        jnp.dot(xn, W1a_ref[...], preferred_element_type=jnp.float32)
        + jnp.dot(hn, W1b_ref[...], preferred_element_type=jnp.float32)
        + bl1_ref[...], 0.0)
    z = jnp.dot(z, W2_ref[...], preferred_element_type=jnp.float32) + bl2_ref[...]
    hnxt = z + h2
    o_ref[...] = (jnp.dot(hnxt, Wo_ref[...], preferred_element_type=jnp.float32)
                  + bo_ref[...])


def _row_spec():
    return pl.BlockSpec((BN, HID), lambda i: (i, 0))


def _full_spec(shape):
    return pl.BlockSpec(shape, lambda i: tuple(0 for _ in shape))


def kernel(x, neighbors, times, rels, start_t, end_t, Wp, bp, ln1_g, ln1_b,
           Wkqv, Wt, bt, Wtime, Wedge, ln2_g, ln2_b, Wl1, bl1, Wl2, bl2,
           Wout, bout):
    f32 = jnp.float32
    st = jnp.asarray(start_t, f32).reshape(1, 1)
    et = jnp.asarray(end_t, f32).reshape(1, 1)

    pad = N_PAD - N
    x_p = jnp.pad(x, ((0, pad), (0, 0)))
    t_p = jnp.pad(times[:, :, 0], ((0, pad), (0, 0)), constant_values=-1.0)
    r_p = jnp.pad(rels.reshape(N, DEG * EDIM), ((0, pad), (0, 0)))
    nb_p = jnp.pad(neighbors.astype(jnp.int32), ((0, pad), (0, 0)))
    Wv = Wkqv[:, 2 * HID:]
    Wtv = Wtime[:, 2 * HID:]
    Wts, Wtc = Wtv[0::2], Wtv[1::2]
    We = Wedge[:, 2 * HID:]
    W1a, W1b = Wl1[:HID], Wl1[HID:]
    bp2 = bp.reshape(1, HID)
    LREP = DEG * (TDIM // 2)
    Rm = jnp.repeat(jnp.eye(DEG, dtype=f32), TDIM // 2, axis=1)
    wt_t = jnp.tile(Wt.reshape(1, TDIM // 2), (1, DEG))
    bt_t = jnp.tile(bt.reshape(1, TDIM // 2), (1, DEG))
    SWs = jnp.tile(Wts, (DEG, 1))
    SWc = jnp.tile(Wtc, (DEG, 1))
    SWe = jnp.tile(We, (DEG, 1))
    g1, b1 = ln1_g.reshape(1, HID), ln1_b.reshape(1, HID)
    g2, b2 = ln2_g.reshape(1, HID), ln2_b.reshape(1, HID)
    bl1r, bl2r = bl1.reshape(1, HID), bl2.reshape(1, HID)
    bor = bout.reshape(1, OUT)

    tbm, idx2d, xn, v = pl.pallas_call(
        _pre_body,
        grid=(GRID,),
        in_specs=[
            _full_spec((1, 1)), _full_spec((1, 1)),
            _row_spec(),
            pl.BlockSpec((BN, DEG), lambda i: (i, 0)),
            pl.BlockSpec((BN, DEG * EDIM), lambda i: (i, 0)),
            pl.BlockSpec((BN, DEG), lambda i: (i, 0)),
            _full_spec((HID, HID)), _full_spec((1, HID)),
            _full_spec((DEG, LREP)),
            _full_spec((1, LREP)), _full_spec((1, LREP)),
            _full_spec((LREP, HID)), _full_spec((LREP, HID)),
            _full_spec((LREP, HID)),
            _full_spec((1, HID)), _full_spec((1, HID)),
            _full_spec((HID, HID)),
        ],
        out_specs=[_row_spec(),
                   pl.BlockSpec((BN, DEG), lambda i: (i, 0)),
                   _row_spec(), _row_spec()],
        out_shape=[
            jax.ShapeDtypeStruct((N_PAD, HID), f32),
            jax.ShapeDtypeStruct((N_PAD, DEG), jnp.int32),
            jax.ShapeDtypeStruct((N_PAD, HID), f32),
            jax.ShapeDtypeStruct((N_PAD, HID), f32),
        ],
    )(st, et, x_p, t_p, r_p, nb_p, Wp, bp2, Rm, wt_t, bt_t, SWs, SWc, SWe,
      g1, b1, Wv)
    idx2d = idx2d.reshape(N_PAD // CN, CN * DEG)

    didx = (jnp.arange(16, dtype=jnp.int32)[:, None, None] * (NBUF * CN)
            + jnp.arange(NBUF, dtype=jnp.int32)[None, :, None] * CN
            + jnp.repeat(jnp.arange(CN, dtype=jnp.int32), DEG)[None, None, :]
            ).reshape(16 * NBUF * CN * DEG)

    mlp_lnv = pl.pallas_call(
        _mlp_lnv_body,
        grid=(GRID,),
        in_specs=[_row_spec(), _row_spec(), _row_spec(),
                  _full_spec((1, HID)), _full_spec((1, HID)),
                  _full_spec((HID, HID)), _full_spec((HID, HID)),
                  _full_spec((1, HID)), _full_spec((HID, HID)),
                  _full_spec((1, HID)),
                  _full_spec((1, HID)), _full_spec((1, HID)),
                  _full_spec((HID, HID))],
        out_specs=[_row_spec(), _row_spec()],
        out_shape=[jax.ShapeDtypeStruct((N_PAD, HID), f32),
                   jax.ShapeDtypeStruct((N_PAD, HID), f32)],
    )

    mlp_final = pl.pallas_call(
        _mlp_final_body,
        grid=(GRID,),
        in_specs=[_row_spec(), _row_spec(), _row_spec(),
                  _full_spec((1, HID)), _full_spec((1, HID)),
                  _full_spec((HID, HID)), _full_spec((HID, HID)),
                  _full_spec((1, HID)), _full_spec((HID, HID)),
                  _full_spec((1, HID)), _full_spec((HID, OUT)),
                  _full_spec((1, OUT))],
        out_specs=pl.BlockSpec((BN, OUT), lambda i: (i, 0)),
        out_shape=jax.ShapeDtypeStruct((N_PAD, OUT), f32),
    )

    agg = _make_gather_sum()(v, idx2d, didx)
    xn, v = mlp_lnv(xn, agg, tbm, g2, b2, W1a, W1b, bl1r, Wl2, bl2r,
                    g1, b1, Wv)
    agg = _make_gather_sum()(v, idx2d, didx)
    out = mlp_final(xn, agg, tbm, g2, b2, W1a, W1b, bl1r, Wl2, bl2r, Wout, bor)
    return out[:N]


# speedup vs baseline: 1.0604x; 1.0589x over previous
import functools

import jax
import jax.numpy as jnp
from jax import lax
from jax.experimental import pallas as pl
from jax.experimental.pallas import tpu as pltpu
from jax.experimental.pallas import tpu_sc as plsc

N = 10000
DEG = 16
HID = 128
TDIM = 32
EDIM = 16
OUT = 128
T_NORM = (1.0 / (TDIM // 2)) ** 0.5
EPS = 1e-5

NW = 32
N_PAD = 10240
NPW = N_PAD // NW
CN = 8
BN = 256
GRID = N_PAD // BN


def _ln(h, g, b):
    m = jnp.mean(h, axis=-1, keepdims=True)
    v = jnp.mean((h - m) ** 2, axis=-1, keepdims=True)
    return (h - m) * jax.lax.rsqrt(v + EPS) * g + b


def _pre1_body(st_ref, et_ref, x_ref, t_ref, nb_ref, Wp_ref, bp_ref,
               g_ref, b_ref, Wv_ref, idx_ref, xn_ref, v_ref):
    st = st_ref[0, 0]
    et = et_ref[0, 0]
    t = t_ref[...]
    mask = (t >= st) & (t < et)
    h0 = jnp.maximum(
        jnp.dot(x_ref[...], Wp_ref[...], preferred_element_type=jnp.float32)
        + bp_ref[...], 0.0)
    xn = _ln(h0, g_ref[...], b_ref[...])
    xn_ref[...] = xn
    v = jnp.dot(xn, Wv_ref[...], preferred_element_type=jnp.float32)
    row = (pl.program_id(0) * BN
           + lax.broadcasted_iota(jnp.int32, (BN, 1), 0))
    v_ref[...] = jnp.where(row < N, v, 0.0)
    idx_ref[...] = jnp.where(mask, nb_ref[...], N)


def _pre2_body(st_ref, et_ref, t_ref, r_ref, R_ref, wt_ref, bt_ref,
               SWs_ref, SWc_ref, SWe_ref, tbm_ref):
    st = st_ref[0, 0]
    et = et_ref[0, 0]
    t = t_ref[...]
    mask = (t >= st) & (t < et)
    maskf = mask.astype(jnp.float32)
    anymask = jnp.max(maskf, axis=1, keepdims=True)
    tmax = jnp.maximum(st, jnp.max(jnp.where(mask, t, -jnp.inf), axis=1,
                                   keepdims=True))
    Rm = R_ref[...]
    t_rep = jnp.dot(t, Rm, preferred_element_type=jnp.float32)
    m_rep = jnp.dot(maskf, Rm, preferred_element_type=jnp.float32)
    hh = (tmax - t_rep) * wt_ref[...] + bt_ref[...]
    tb = (jnp.dot(m_rep * jnp.sin(hh), SWs_ref[...],
                  preferred_element_type=jnp.float32)
          + jnp.dot(m_rep * jnp.cos(hh), SWc_ref[...],
                    preferred_element_type=jnp.float32)) * T_NORM
    tb = tb + jnp.dot(m_rep * r_ref[...], SWe_ref[...],
                      preferred_element_type=jnp.float32)
    tbm_ref[...] = anymask * tb


def _mlp_lnv_body(xn_ref, agg_ref, tbm_ref, g2_ref, b2_ref, W1a_ref, W1b_ref,
                  bl1_ref, W2_ref, bl2_ref, g1_ref, b1_ref, Wv_ref,
                  xn2_ref, v_ref):
    xn = xn_ref[...]
    h2 = agg_ref[...] + tbm_ref[...] + xn
    hn = _ln(h2, g2_ref[...], b2_ref[...])
    z = jnp.maximum(
        jnp.dot(xn, W1a_ref[...], preferred_element_type=jnp.float32)
        + jnp.dot(hn, W1b_ref[...], preferred_element_type=jnp.float32)
        + bl1_ref[...], 0.0)
    z = jnp.dot(z, W2_ref[...], preferred_element_type=jnp.float32) + bl2_ref[...]
    h = z + h2
    xn2 = _ln(h, g1_ref[...], b1_ref[...])
    xn2_ref[...] = xn2
    v = jnp.dot(xn2, Wv_ref[...], preferred_element_type=jnp.float32)
    row = (pl.program_id(0) * BN
           + lax.broadcasted_iota(jnp.int32, (BN, 1), 0))
    v_ref[...] = jnp.where(row < N, v, 0.0)


NCH = NPW // CN
NBUF = 4


@functools.cache
def _make_gather_sum():
    mesh = plsc.VectorSubcoreMesh(core_axis_name="c", subcore_axis_name="s")

    @functools.partial(
        pl.kernel,
        out_type=jax.ShapeDtypeStruct((N_PAD, HID), jnp.float32),
        mesh=mesh,
        scratch_types=[
            pltpu.VMEM((NCH, CN * DEG), jnp.int32),
            pltpu.VMEM((CN * DEG,), jnp.int32),
            pltpu.VMEM((CN * DEG,), jnp.int32),
            pltpu.VMEM((CN * DEG,), jnp.int32),
            pltpu.VMEM((CN * DEG,), jnp.int32),
            pltpu.VMEM((CN, HID), jnp.float32),
            pltpu.VMEM((NBUF, CN * DEG, HID), jnp.float32),
            pltpu.VMEM_SHARED((16 * NBUF * CN, HID), jnp.float32),
            pltpu.SemaphoreType.DMA,
            pltpu.SemaphoreType.DMA,
            pltpu.SemaphoreType.DMA,
            pltpu.SemaphoreType.DMA,
            pltpu.SemaphoreType.DMA,
            pltpu.SemaphoreType.DMA,
            pltpu.SemaphoreType.DMA,
            pltpu.SemaphoreType.DMA,
        ],
    )
    def _gather_sum(v_hbm, idx_hbm, didx_hbm, out_hbm, idx_s, d0, d1, d2, d3,
                    zero_s, rows_s, acc_sh, g0, g1, g2, g3, o0, o1, o2, o3):
        didx_s = (d0, d1, d2, d3)
        gsem = (g0, g1, g2, g3)
        osem = (o0, o1, o2, o3)
        sid = lax.axis_index("s")
        wid = sid * 2 + lax.axis_index("c")
        base = wid * NPW
        pltpu.sync_copy(idx_hbm.at[pl.ds(wid * NCH, NCH)], idx_s)
        for b in range(NBUF):
            pltpu.sync_copy(
                didx_hbm.at[pl.ds((sid * NBUF + b) * (CN * DEG), CN * DEG)],
                didx_s[b])
        for c in range(CN):
            for j in range(HID // 16):
                zero_s[c, pl.ds(j * 16, 16)] = jnp.zeros((16,), jnp.float32)

        def acc_rows(b):
            return acc_sh.at[pl.ds(sid * (NBUF * CN) + b * CN, CN)]

        def issue_gather(ci, b):
            return pltpu.async_copy(v_hbm.at[idx_s.at[ci]], rows_s.at[b],
                                    gsem[b])

        for b in range(NBUF):
            issue_gather(b, b)

        def group(g, carry):
            for b in range(NBUF):
                ci = g * NBUF + b
                node0 = base + ci * CN
                pltpu.make_async_copy(v_hbm.at[idx_s.at[ci]], rows_s.at[b],
                                      gsem[b]).wait()

                @pl.when(g > 0)
                def _wait_out():
                    pltpu.make_async_copy(
                        acc_rows(b), out_hbm.at[pl.ds(node0, CN)],
                        osem[b]).wait()

                pltpu.sync_copy(zero_s, acc_rows(b))
                pltpu.sync_copy(rows_s.at[b], acc_sh.at[didx_s[b]],
                                add=True)
                nc = ci + NBUF

                @pl.when(nc < NCH)
                def _next():
                    issue_gather(nc, b)

            plsc.subcore_barrier()
            for b in range(NBUF):
                node0 = base + (g * NBUF + b) * CN
                pltpu.async_copy(acc_rows(b), out_hbm.at[pl.ds(node0, CN)],
                                 osem[b])
            return carry

        lax.fori_loop(0, NCH // NBUF, group, 0)
        for b in range(NBUF):
            node0 = base + (NCH - NBUF + b) * CN
            pltpu.make_async_copy(acc_rows(b), out_hbm.at[pl.ds(node0, CN)],
                                  osem[b]).wait()

    return _gather_sum


def _mlp_final_body(xn_ref, agg_ref, tbm_ref, g_ref, b_ref, W1a_ref, W1b_ref,
                    bl1_ref, W2_ref, bl2_ref, Wo_ref, bo_ref, o_ref):
    xn = xn_ref[...]
    h2 = agg_ref[...] + tbm_ref[...] + xn
    hn = _ln(h2, g_ref[...], b_ref[...])
    z = jnp.maximum(
        jnp.dot(xn, W1a_ref[...], preferred_element_type=jnp.float32)
        + jnp.dot(hn, W1b_ref[...], preferred_element_type=jnp.float32)
        + bl1_ref[...], 0.0)
    z = jnp.dot(z, W2_ref[...], preferred_element_type=jnp.float32) + bl2_ref[...]
    hnxt = z + h2
    o_ref[...] = (jnp.dot(hnxt, Wo_ref[...], preferred_element_type=jnp.float32)
                  + bo_ref[...])


def _row_spec():
    return pl.BlockSpec((BN, HID), lambda i: (i, 0))


def _full_spec(shape):
    return pl.BlockSpec(shape, lambda i: tuple(0 for _ in shape))


def kernel(x, neighbors, times, rels, start_t, end_t, Wp, bp, ln1_g, ln1_b,
           Wkqv, Wt, bt, Wtime, Wedge, ln2_g, ln2_b, Wl1, bl1, Wl2, bl2,
           Wout, bout):
    f32 = jnp.float32
    st = jnp.asarray(start_t, f32).reshape(1, 1)
    et = jnp.asarray(end_t, f32).reshape(1, 1)

    pad = N_PAD - N
    x_p = jnp.pad(x, ((0, pad), (0, 0)))
    t_p = jnp.pad(times[:, :, 0], ((0, pad), (0, 0)), constant_values=-1.0)
    r_p = jnp.pad(rels.reshape(N, DEG * EDIM), ((0, pad), (0, 0)))
    nb_p = jnp.pad(neighbors.astype(jnp.int32), ((0, pad), (0, 0)))
    Wv = Wkqv[:, 2 * HID:]
    Wtv = Wtime[:, 2 * HID:]
    Wts, Wtc = Wtv[0::2], Wtv[1::2]
    We = Wedge[:, 2 * HID:]
    W1a, W1b = Wl1[:HID], Wl1[HID:]
    bp2 = bp.reshape(1, HID)
    LREP = DEG * (TDIM // 2)
    Rm = jnp.repeat(jnp.eye(DEG, dtype=f32), TDIM // 2, axis=1)
    wt_t = jnp.tile(Wt.reshape(1, TDIM // 2), (1, DEG))
    bt_t = jnp.tile(bt.reshape(1, TDIM // 2), (1, DEG))
    SWs = jnp.tile(Wts, (DEG, 1))
    SWc = jnp.tile(Wtc, (DEG, 1))
    SWe = jnp.tile(We, (DEG, 1))
    g1, b1 = ln1_g.reshape(1, HID), ln1_b.reshape(1, HID)
    g2, b2 = ln2_g.reshape(1, HID), ln2_b.reshape(1, HID)
    bl1r, bl2r = bl1.reshape(1, HID), bl2.reshape(1, HID)
    bor = bout.reshape(1, OUT)

    idx2d, xn, v = pl.pallas_call(
        _pre1_body,
        grid=(GRID,),
        in_specs=[
            _full_spec((1, 1)), _full_spec((1, 1)),
            _row_spec(),
            pl.BlockSpec((BN, DEG), lambda i: (i, 0)),
            pl.BlockSpec((BN, DEG), lambda i: (i, 0)),
            _full_spec((HID, HID)), _full_spec((1, HID)),
            _full_spec((1, HID)), _full_spec((1, HID)),
            _full_spec((HID, HID)),
        ],
        out_specs=[pl.BlockSpec((BN, DEG), lambda i: (i, 0)),
                   _row_spec(), _row_spec()],
        out_shape=[
            jax.ShapeDtypeStruct((N_PAD, DEG), jnp.int32),
            jax.ShapeDtypeStruct((N_PAD, HID), f32),
            jax.ShapeDtypeStruct((N_PAD, HID), f32),
        ],
    )(st, et, x_p, t_p, nb_p, Wp, bp2, g1, b1, Wv)
    idx2d = idx2d.reshape(N_PAD // CN, CN * DEG)

    tbm = pl.pallas_call(
        _pre2_body,
        grid=(GRID,),
        in_specs=[
            _full_spec((1, 1)), _full_spec((1, 1)),
            pl.BlockSpec((BN, DEG), lambda i: (i, 0)),
            pl.BlockSpec((BN, DEG * EDIM), lambda i: (i, 0)),
            _full_spec((DEG, LREP)),
            _full_spec((1, LREP)), _full_spec((1, LREP)),
            _full_spec((LREP, HID)), _full_spec((LREP, HID)),
            _full_spec((LREP, HID)),
        ],
        out_specs=_row_spec(),
        out_shape=jax.ShapeDtypeStruct((N_PAD, HID), f32),
    )(st, et, t_p, r_p, Rm, wt_t, bt_t, SWs, SWc, SWe)

    didx = (jnp.arange(16, dtype=jnp.int32)[:, None, None] * (NBUF * CN)
            + jnp.arange(NBUF, dtype=jnp.int32)[None, :, None] * CN
            + jnp.repeat(jnp.arange(CN, dtype=jnp.int32), DEG)[None, None, :]
            ).reshape(16 * NBUF * CN * DEG)

    mlp_lnv = pl.pallas_call(
        _mlp_lnv_body,
        grid=(GRID,),
        in_specs=[_row_spec(), _row_spec(), _row_spec(),
                  _full_spec((1, HID)), _full_spec((1, HID)),
                  _full_spec((HID, HID)), _full_spec((HID, HID)),
                  _full_spec((1, HID)), _full_spec((HID, HID)),
                  _full_spec((1, HID)),
                  _full_spec((1, HID)), _full_spec((1, HID)),
                  _full_spec((HID, HID))],
        out_specs=[_row_spec(), _row_spec()],
        out_shape=[jax.ShapeDtypeStruct((N_PAD, HID), f32),
                   jax.ShapeDtypeStruct((N_PAD, HID), f32)],
    )

    mlp_final = pl.pallas_call(
        _mlp_final_body,
        grid=(GRID,),
        in_specs=[_row_spec(), _row_spec(), _row_spec(),
                  _full_spec((1, HID)), _full_spec((1, HID)),
                  _full_spec((HID, HID)), _full_spec((HID, HID)),
                  _full_spec((1, HID)), _full_spec((HID, HID)),
                  _full_spec((1, HID)), _full_spec((HID, OUT)),
                  _full_spec((1, OUT))],
        out_specs=pl.BlockSpec((BN, OUT), lambda i: (i, 0)),
        out_shape=jax.ShapeDtypeStruct((N_PAD, OUT), f32),
    )

    agg = _make_gather_sum()(v, idx2d, didx)
    xn, v = mlp_lnv(xn, agg, tbm, g2, b2, W1a, W1b, bl1r, Wl2, bl2r,
                    g1, b1, Wv)
    agg = _make_gather_sum()(v, idx2d, didx)
    out = mlp_final(xn, agg, tbm, g2, b2, W1a, W1b, bl1r, Wl2, bl2r, Wout, bor)
    return out[:N]


# speedup vs baseline: 1.1939x; 1.1259x over previous
import functools

import jax
import jax.numpy as jnp
from jax import lax
from jax.experimental import pallas as pl
from jax.experimental.pallas import tpu as pltpu
from jax.experimental.pallas import tpu_sc as plsc

N = 10000
DEG = 16
HID = 128
TDIM = 32
EDIM = 16
OUT = 128
T_NORM = (1.0 / (TDIM // 2)) ** 0.5
EPS = 1e-5

NW = 32
N_PAD = 10240
NPW = N_PAD // NW
CN = 8
BN = 256
GRID = N_PAD // BN


def _ln(h, g, b):
    m = jnp.mean(h, axis=-1, keepdims=True)
    v = jnp.mean((h - m) ** 2, axis=-1, keepdims=True)
    return (h - m) * jax.lax.rsqrt(v + EPS) * g + b


def _pre1_body(st_ref, et_ref, x_ref, t_ref, nb_ref, Wp_ref, bp_ref,
               g_ref, b_ref, Wv_ref, idx_ref, xn_ref, v_ref):
    st = st_ref[0, 0]
    et = et_ref[0, 0]
    t = t_ref[...]
    mask = (t >= st) & (t < et)
    h0 = jnp.maximum(
        jnp.dot(x_ref[...], Wp_ref[...], preferred_element_type=jnp.float32)
        + bp_ref[...], 0.0)
    xn = _ln(h0, g_ref[...], b_ref[...])
    xn_ref[...] = xn
    v = jnp.dot(xn, Wv_ref[...], preferred_element_type=jnp.float32)
    row = (pl.program_id(0) * BN
           + lax.broadcasted_iota(jnp.int32, (BN, 1), 0))
    v_ref[...] = jnp.where(row < N, v, 0.0)
    idx_ref[...] = jnp.where(mask, nb_ref[...], N)


def _pre2_body(st_ref, et_ref, t_ref, r_ref, R_ref, wt_ref, bt_ref,
               SWs_ref, SWc_ref, SWe_ref, tbm_ref):
    st = st_ref[0, 0]
    et = et_ref[0, 0]
    t = t_ref[...]
    mask = (t >= st) & (t < et)
    maskf = mask.astype(jnp.float32)
    anymask = jnp.max(maskf, axis=1, keepdims=True)
    tmax = jnp.maximum(st, jnp.max(jnp.where(mask, t, -jnp.inf), axis=1,
                                   keepdims=True))
    Rm = R_ref[...]
    t_rep = jnp.dot(t, Rm, preferred_element_type=jnp.float32)
    m_rep = jnp.dot(maskf, Rm, preferred_element_type=jnp.float32)
    hh = (tmax - t_rep) * wt_ref[...] + bt_ref[...]
    tb = (jnp.dot(m_rep * jnp.sin(hh), SWs_ref[...],
                  preferred_element_type=jnp.float32)
          + jnp.dot(m_rep * jnp.cos(hh), SWc_ref[...],
                    preferred_element_type=jnp.float32)) * T_NORM
    tb = tb + jnp.dot(m_rep * r_ref[...], SWe_ref[...],
                      preferred_element_type=jnp.float32)
    tbm_ref[...] = anymask * tb


def _mlp_lnv_body(xn_ref, agg_ref, tbm_ref, g2_ref, b2_ref, W1a_ref, W1b_ref,
                  bl1_ref, W2_ref, bl2_ref, g1_ref, b1_ref, Wv_ref,
                  xn2_ref, v_ref):
    xn = xn_ref[...]
    h2 = agg_ref[...] + tbm_ref[...] + xn
    hn = _ln(h2, g2_ref[...], b2_ref[...])
    z = jnp.maximum(
        jnp.dot(xn, W1a_ref[...], preferred_element_type=jnp.float32)
        + jnp.dot(hn, W1b_ref[...], preferred_element_type=jnp.float32)
        + bl1_ref[...], 0.0)
    z = jnp.dot(z, W2_ref[...], preferred_element_type=jnp.float32) + bl2_ref[...]
    h = z + h2
    xn2 = _ln(h, g1_ref[...], b1_ref[...])
    xn2_ref[...] = xn2
    v = jnp.dot(xn2, Wv_ref[...], preferred_element_type=jnp.float32)
    row = (pl.program_id(0) * BN
           + lax.broadcasted_iota(jnp.int32, (BN, 1), 0))
    v_ref[...] = jnp.where(row < N, v, 0.0)


NCH = NPW // CN
NBUF = 4


@functools.cache
def _make_gather_sum():
    mesh = plsc.VectorSubcoreMesh(core_axis_name="c", subcore_axis_name="s")

    @functools.partial(
        pl.kernel,
        out_type=jax.ShapeDtypeStruct((N_PAD, HID), jnp.float32),
        mesh=mesh,
        scratch_types=[
            pltpu.VMEM((NCH, CN * DEG), jnp.int32),
            pltpu.VMEM((CN * DEG,), jnp.int32),
            pltpu.VMEM((CN * DEG,), jnp.int32),
            pltpu.VMEM((CN * DEG,), jnp.int32),
            pltpu.VMEM((CN * DEG,), jnp.int32),
            pltpu.VMEM((CN, HID), jnp.float32),
            pltpu.VMEM((NBUF, CN * DEG, HID), jnp.float32),
            pltpu.VMEM_SHARED((16 * NBUF * CN, HID), jnp.float32),
            pltpu.SemaphoreType.DMA,
            pltpu.SemaphoreType.DMA,
            pltpu.SemaphoreType.DMA,
            pltpu.SemaphoreType.DMA,
            pltpu.SemaphoreType.DMA,
            pltpu.SemaphoreType.DMA,
            pltpu.SemaphoreType.DMA,
            pltpu.SemaphoreType.DMA,
        ],
    )
    def _gather_sum(v_hbm, idx_hbm, didx_hbm, out_hbm, idx_s, d0, d1, d2, d3,
                    zero_s, rows_s, acc_sh, g0, g1, g2, g3, o0, o1, o2, o3):
        didx_s = (d0, d1, d2, d3)
        gsem = (g0, g1, g2, g3)
        osem = (o0, o1, o2, o3)
        sid = lax.axis_index("s")
        wid = sid * 2 + lax.axis_index("c")
        base = wid * NPW
        pltpu.sync_copy(idx_hbm.at[pl.ds(wid * NCH, NCH)], idx_s)
        for b in range(NBUF):
            pltpu.sync_copy(
                didx_hbm.at[pl.ds((sid * NBUF + b) * (CN * DEG), CN * DEG)],
                didx_s[b])
        for c in range(CN):
            for j in range(HID // 16):
                zero_s[c, pl.ds(j * 16, 16)] = jnp.zeros((16,), jnp.float32)

        def acc_rows(b):
            return acc_sh.at[pl.ds(sid * (NBUF * CN) + b * CN, CN)]

        def issue_gather(ci, b):
            return pltpu.async_copy(v_hbm.at[idx_s.at[ci]], rows_s.at[b],
                                    gsem[b])

        for b in range(NBUF):
            issue_gather(b, b)

        def group(g, carry):
            for b in range(NBUF):
                ci = g * NBUF + b
                node0 = base + ci * CN
                pltpu.make_async_copy(v_hbm.at[idx_s.at[ci]], rows_s.at[b],
                                      gsem[b]).wait()

                @pl.when(g > 0)
                def _wait_out():
                    pltpu.make_async_copy(
                        acc_rows(b), out_hbm.at[pl.ds(node0, CN)],
                        osem[b]).wait()

                pltpu.sync_copy(zero_s, acc_rows(b))
                pltpu.sync_copy(rows_s.at[b], acc_sh.at[didx_s[b]],
                                add=True)
                nc = ci + NBUF

                @pl.when(nc < NCH)
                def _next():
                    issue_gather(nc, b)

            plsc.subcore_barrier()
            for b in range(NBUF):
                node0 = base + (g * NBUF + b) * CN
                pltpu.async_copy(acc_rows(b), out_hbm.at[pl.ds(node0, CN)],
                                 osem[b])
            return carry

        lax.fori_loop(0, NCH // NBUF, group, 0)
        for b in range(NBUF):
            node0 = base + (NCH - NBUF + b) * CN
            pltpu.make_async_copy(acc_rows(b), out_hbm.at[pl.ds(node0, CN)],
                                  osem[b]).wait()

    return _gather_sum


def _mlp_final_body(xn_ref, agg_ref, tbm_ref, g_ref, b_ref, W1a_ref, W1b_ref,
                    bl1_ref, W2_ref, bl2_ref, Wo_ref, bo_ref, o_ref):
    xn = xn_ref[...]
    h2 = agg_ref[...] + tbm_ref[...] + xn
    hn = _ln(h2, g_ref[...], b_ref[...])
    z = jnp.maximum(
        jnp.dot(xn, W1a_ref[...], preferred_element_type=jnp.float32)
        + jnp.dot(hn, W1b_ref[...], preferred_element_type=jnp.float32)
        + bl1_ref[...], 0.0)
    z = jnp.dot(z, W2_ref[...], preferred_element_type=jnp.float32) + bl2_ref[...]
    hnxt = z + h2
    o_ref[...] = (jnp.dot(hnxt, Wo_ref[...], preferred_element_type=jnp.float32)
                  + bo_ref[...])


def _row_spec():
    return pl.BlockSpec((BN, HID), lambda i: (i, 0))


def _full_spec(shape):
    return pl.BlockSpec(shape, lambda i: tuple(0 for _ in shape))


def kernel(x, neighbors, times, rels, start_t, end_t, Wp, bp, ln1_g, ln1_b,
           Wkqv, Wt, bt, Wtime, Wedge, ln2_g, ln2_b, Wl1, bl1, Wl2, bl2,
           Wout, bout):
    f32 = jnp.float32
    st = jnp.asarray(start_t, f32).reshape(1, 1)
    et = jnp.asarray(end_t, f32).reshape(1, 1)

    pad = N_PAD - N
    x_p = jnp.pad(x, ((0, pad), (0, 0)))
    t_p = jnp.pad(times[:, :, 0], ((0, pad), (0, 0)), constant_values=-1.0)
    r_p = jnp.pad(rels.reshape(N, DEG * EDIM), ((0, pad), (0, 0)))
    nb_p = jnp.pad(neighbors.astype(jnp.int32), ((0, pad), (0, 0)))
    Wv = Wkqv[:, 2 * HID:]
    Wtv = Wtime[:, 2 * HID:]
    Wts, Wtc = Wtv[0::2], Wtv[1::2]
    We = Wedge[:, 2 * HID:]
    W1a, W1b = Wl1[:HID], Wl1[HID:]
    bp2 = bp.reshape(1, HID)
    LREP = DEG * (TDIM // 2)
    Rm = jnp.repeat(jnp.eye(DEG, dtype=f32), TDIM // 2, axis=1)
    wt_t = jnp.tile(Wt.reshape(1, TDIM // 2), (1, DEG))
    bt_t = jnp.tile(bt.reshape(1, TDIM // 2), (1, DEG))
    SWs = jnp.tile(Wts, (DEG, 1))
    SWc = jnp.tile(Wtc, (DEG, 1))
    SWe = jnp.tile(We, (DEG, 1))
    g1, b1 = ln1_g.reshape(1, HID), ln1_b.reshape(1, HID)
    g2, b2 = ln2_g.reshape(1, HID), ln2_b.reshape(1, HID)
    bl1r, bl2r = bl1.reshape(1, HID), bl2.reshape(1, HID)
    bor = bout.reshape(1, OUT)

    idx2d, xn, v = pl.pallas_call(
        _pre1_body,
        grid=(GRID,),
        in_specs=[
            _full_spec((1, 1)), _full_spec((1, 1)),
            _row_spec(),
            pl.BlockSpec((BN // CN, CN * DEG), lambda i: (i, 0)),
            pl.BlockSpec((BN // CN, CN * DEG), lambda i: (i, 0)),
            _full_spec((HID, HID)), _full_spec((1, HID)),
            _full_spec((1, HID)), _full_spec((1, HID)),
            _full_spec((HID, HID)),
        ],
        out_specs=[pl.BlockSpec((BN // CN, CN * DEG), lambda i: (i, 0)),
                   _row_spec(), _row_spec()],
        out_shape=[
            jax.ShapeDtypeStruct((N_PAD // CN, CN * DEG), jnp.int32),
            jax.ShapeDtypeStruct((N_PAD, HID), f32),
            jax.ShapeDtypeStruct((N_PAD, HID), f32),
        ],
    )(st, et, x_p, t_p.reshape(N_PAD // CN, CN * DEG),
      nb_p.reshape(N_PAD // CN, CN * DEG), Wp, bp2, g1, b1, Wv)

    tbm = pl.pallas_call(
        _pre2_body,
        grid=(GRID,),
        in_specs=[
            _full_spec((1, 1)), _full_spec((1, 1)),
            pl.BlockSpec((BN, DEG), lambda i: (i, 0)),
            pl.BlockSpec((BN, DEG * EDIM), lambda i: (i, 0)),
            _full_spec((DEG, LREP)),
            _full_spec((1, LREP)), _full_spec((1, LREP)),
            _full_spec((LREP, HID)), _full_spec((LREP, HID)),
            _full_spec((LREP, HID)),
        ],
        out_specs=_row_spec(),
        out_shape=jax.ShapeDtypeStruct((N_PAD, HID), f32),
    )(st, et, t_p, r_p, Rm, wt_t, bt_t, SWs, SWc, SWe)

    didx = (jnp.arange(16, dtype=jnp.int32)[:, None, None] * (NBUF * CN)
            + jnp.arange(NBUF, dtype=jnp.int32)[None, :, None] * CN
            + jnp.repeat(jnp.arange(CN, dtype=jnp.int32), DEG)[None, None, :]
            ).reshape(16 * NBUF * CN * DEG)

    mlp_lnv = pl.pallas_call(
        _mlp_lnv_body,
        grid=(GRID,),
        in_specs=[_row_spec(), _row_spec(), _row_spec(),
                  _full_spec((1, HID)), _full_spec((1, HID)),
                  _full_spec((HID, HID)), _full_spec((HID, HID)),
                  _full_spec((1, HID)), _full_spec((HID, HID)),
                  _full_spec((1, HID)),
                  _full_spec((1, HID)), _full_spec((1, HID)),
                  _full_spec((HID, HID))],
        out_specs=[_row_spec(), _row_spec()],
        out_shape=[jax.ShapeDtypeStruct((N_PAD, HID), f32),
                   jax.ShapeDtypeStruct((N_PAD, HID), f32)],
    )

    mlp_final = pl.pallas_call(
        _mlp_final_body,
        grid=(GRID,),
        in_specs=[_row_spec(), _row_spec(), _row_spec(),
                  _full_spec((1, HID)), _full_spec((1, HID)),
                  _full_spec((HID, HID)), _full_spec((HID, HID)),
                  _full_spec((1, HID)), _full_spec((HID, HID)),
                  _full_spec((1, HID)), _full_spec((HID, OUT)),
                  _full_spec((1, OUT))],
        out_specs=pl.BlockSpec((BN, OUT), lambda i: (i, 0)),
        out_shape=jax.ShapeDtypeStruct((N_PAD, OUT), f32),
    )

    agg = _make_gather_sum()(v, idx2d, didx)
    xn, v = mlp_lnv(xn, agg, tbm, g2, b2, W1a, W1b, bl1r, Wl2, bl2r,
                    g1, b1, Wv)
    agg = _make_gather_sum()(v, idx2d, didx)
    out = mlp_final(xn, agg, tbm, g2, b2, W1a, W1b, bl1r, Wl2, bl2r, Wout, bor)
    return out[:N]


# speedup vs baseline: 1.2013x; 1.0062x over previous
import functools

import jax
import jax.numpy as jnp
from jax import lax
from jax.experimental import pallas as pl
from jax.experimental.pallas import tpu as pltpu
from jax.experimental.pallas import tpu_sc as plsc

N = 10000
DEG = 16
HID = 128
TDIM = 32
EDIM = 16
OUT = 128
T_NORM = (1.0 / (TDIM // 2)) ** 0.5
EPS = 1e-5

NW = 32
N_PAD = 10240
NPW = N_PAD // NW
CN = 8
BN = 256
GRID = N_PAD // BN


def _ln(h, g, b):
    m = jnp.mean(h, axis=-1, keepdims=True)
    v = jnp.mean((h - m) ** 2, axis=-1, keepdims=True)
    return (h - m) * jax.lax.rsqrt(v + EPS) * g + b


def _pre1_body(st_ref, et_ref, x_ref, t_ref, nb_ref, Wp_ref, bp_ref,
               g_ref, b_ref, Wv_ref, idx_ref, xn_ref, v_ref):
    st = st_ref[0, 0]
    et = et_ref[0, 0]
    t = t_ref[...]
    mask = (t >= st) & (t < et)
    h0 = jnp.maximum(
        jnp.dot(x_ref[...], Wp_ref[...], preferred_element_type=jnp.float32)
        + bp_ref[...], 0.0)
    xn = _ln(h0, g_ref[...], b_ref[...])
    xn_ref[...] = xn
    v = jnp.dot(xn, Wv_ref[...], preferred_element_type=jnp.float32)
    row = (pl.program_id(0) * BN
           + lax.broadcasted_iota(jnp.int32, (BN, 1), 0))
    v_ref[...] = jnp.where(row < N, v, 0.0)
    idx_ref[...] = jnp.where(mask, nb_ref[...], N)


def _pre2_body(st_ref, et_ref, t_ref, r_ref, R_ref, wt_ref, bt_ref,
               SWs_ref, SWc_ref, SWe_ref, tbm_ref):
    st = st_ref[0, 0]
    et = et_ref[0, 0]
    t = t_ref[...]
    mask = (t >= st) & (t < et)
    maskf = mask.astype(jnp.float32)
    anymask = jnp.max(maskf, axis=1, keepdims=True)
    tmax = jnp.maximum(st, jnp.max(jnp.where(mask, t, -jnp.inf), axis=1,
                                   keepdims=True))
    Rm = R_ref[...]
    t_rep = jnp.dot(t, Rm, preferred_element_type=jnp.float32)
    m_rep = jnp.dot(maskf, Rm, preferred_element_type=jnp.float32)
    hh = (tmax - t_rep) * wt_ref[...] + bt_ref[...]
    tb = (jnp.dot(m_rep * jnp.sin(hh), SWs_ref[...],
                  preferred_element_type=jnp.float32)
          + jnp.dot(m_rep * jnp.cos(hh), SWc_ref[...],
                    preferred_element_type=jnp.float32)) * T_NORM
    tb = tb + jnp.dot(m_rep * r_ref[...], SWe_ref[...],
                      preferred_element_type=jnp.float32)
    tbm_ref[...] = anymask * tb


def _mlp_lnv_body(xn_ref, agg_ref, tbm_ref, g2_ref, b2_ref, W1a_ref, W1b_ref,
                  bl1_ref, W2_ref, bl2_ref, g1_ref, b1_ref, Wv_ref,
                  xn2_ref, v_ref):
    xn = xn_ref[...]
    h2 = agg_ref[...] + tbm_ref[...] + xn
    hn = _ln(h2, g2_ref[...], b2_ref[...])
    z = jnp.maximum(
        jnp.dot(xn, W1a_ref[...], preferred_element_type=jnp.float32)
        + jnp.dot(hn, W1b_ref[...], preferred_element_type=jnp.float32)
        + bl1_ref[...], 0.0)
    z = jnp.dot(z, W2_ref[...], preferred_element_type=jnp.float32) + bl2_ref[...]
    h = z + h2
    xn2 = _ln(h, g1_ref[...], b1_ref[...])
    xn2_ref[...] = xn2
    v = jnp.dot(xn2, Wv_ref[...], preferred_element_type=jnp.float32)
    row = (pl.program_id(0) * BN
           + lax.broadcasted_iota(jnp.int32, (BN, 1), 0))
    v_ref[...] = jnp.where(row < N, v, 0.0)


NCH = NPW // CN
NBUF = 4


@functools.cache
def _make_gather_sum():
    mesh = plsc.VectorSubcoreMesh(core_axis_name="c", subcore_axis_name="s")

    @functools.partial(
        pl.kernel,
        out_type=jax.ShapeDtypeStruct((N_PAD, HID), jnp.float32),
        mesh=mesh,
        scratch_types=[
            pltpu.VMEM((NCH, CN * DEG), jnp.int32),
            pltpu.VMEM((CN * DEG,), jnp.int32),
            pltpu.VMEM((CN * DEG,), jnp.int32),
            pltpu.VMEM((CN * DEG,), jnp.int32),
            pltpu.VMEM((CN * DEG,), jnp.int32),
            pltpu.VMEM((CN, HID), jnp.float32),
            pltpu.VMEM((NBUF, CN * DEG, HID), jnp.float32),
            pltpu.VMEM_SHARED((16 * NBUF * CN, HID), jnp.float32),
            pltpu.SemaphoreType.DMA,
            pltpu.SemaphoreType.DMA,
            pltpu.SemaphoreType.DMA,
            pltpu.SemaphoreType.DMA,
            pltpu.SemaphoreType.DMA,
            pltpu.SemaphoreType.DMA,
            pltpu.SemaphoreType.DMA,
            pltpu.SemaphoreType.DMA,
        ],
    )
    def _gather_sum(v_hbm, idx_hbm, didx_hbm, out_hbm, idx_s, d0, d1, d2, d3,
                    zero_s, rows_s, acc_sh, g0, g1, g2, g3, o0, o1, o2, o3):
        didx_s = (d0, d1, d2, d3)
        gsem = (g0, g1, g2, g3)
        osem = (o0, o1, o2, o3)
        sid = lax.axis_index("s")
        wid = sid * 2 + lax.axis_index("c")
        base = wid * NPW
        pltpu.sync_copy(idx_hbm.at[pl.ds(wid * NCH, NCH)], idx_s)
        for b in range(NBUF):
            pltpu.sync_copy(
                didx_hbm.at[pl.ds((sid * NBUF + b) * (CN * DEG), CN * DEG)],
                didx_s[b])
        for c in range(CN):
            for j in range(HID // 16):
                zero_s[c, pl.ds(j * 16, 16)] = jnp.zeros((16,), jnp.float32)

        def acc_rows(b):
            return acc_sh.at[pl.ds(sid * (NBUF * CN) + b * CN, CN)]

        def issue_gather(ci, b):
            return pltpu.async_copy(v_hbm.at[idx_s.at[ci]], rows_s.at[b],
                                    gsem[b])

        for b in range(NBUF):
            issue_gather(b, b)

        def group(g, carry):
            for b in range(NBUF):
                ci = g * NBUF + b
                node0 = base + ci * CN
                pltpu.make_async_copy(v_hbm.at[idx_s.at[ci]], rows_s.at[b],
                                      gsem[b]).wait()

                @pl.when(g > 0)
                def _wait_out():
                    pltpu.make_async_copy(
                        acc_rows(b), out_hbm.at[pl.ds(node0, CN)],
                        osem[b]).wait()

                pltpu.sync_copy(zero_s, acc_rows(b))
                pltpu.sync_copy(rows_s.at[b], acc_sh.at[didx_s[b]],
                                add=True)
                nc = ci + NBUF

                @pl.when(nc < NCH)
                def _next():
                    issue_gather(nc, b)

            plsc.subcore_barrier()
            for b in range(NBUF):
                node0 = base + (g * NBUF + b) * CN
                pltpu.async_copy(acc_rows(b), out_hbm.at[pl.ds(node0, CN)],
                                 osem[b])
            return carry

        lax.fori_loop(0, NCH // NBUF, group, 0)
        for b in range(NBUF):
            node0 = base + (NCH - NBUF + b) * CN
            pltpu.make_async_copy(acc_rows(b), out_hbm.at[pl.ds(node0, CN)],
                                  osem[b]).wait()

    return _gather_sum


def _mlp_final_body(xn_ref, agg_ref, tbm_ref, g_ref, b_ref, W1a_ref, W1b_ref,
                    bl1_ref, W2_ref, bl2_ref, Wo_ref, bo_ref, o_ref):
    xn = xn_ref[...]
    h2 = agg_ref[...] + tbm_ref[...] + xn
    hn = _ln(h2, g_ref[...], b_ref[...])
    z = jnp.maximum(
        jnp.dot(xn, W1a_ref[...], preferred_element_type=jnp.float32)
        + jnp.dot(hn, W1b_ref[...], preferred_element_type=jnp.float32)
        + bl1_ref[...], 0.0)
    z = jnp.dot(z, W2_ref[...], preferred_element_type=jnp.float32) + bl2_ref[...]
    hnxt = z + h2
    o_ref[...] = (jnp.dot(hnxt, Wo_ref[...], preferred_element_type=jnp.float32)
                  + bo_ref[...])


def _row_spec():
    return pl.BlockSpec((BN, HID), lambda i: (i, 0))


def _full_spec(shape):
    return pl.BlockSpec(shape, lambda i: tuple(0 for _ in shape))


def kernel(x, neighbors, times, rels, start_t, end_t, Wp, bp, ln1_g, ln1_b,
           Wkqv, Wt, bt, Wtime, Wedge, ln2_g, ln2_b, Wl1, bl1, Wl2, bl2,
           Wout, bout):
    f32 = jnp.float32
    st = jnp.asarray(start_t, f32).reshape(1, 1)
    et = jnp.asarray(end_t, f32).reshape(1, 1)

    pad = N_PAD - N
    x_p = jnp.pad(x, ((0, pad), (0, 0)))
    t_p = jnp.pad(times[:, :, 0], ((0, pad), (0, 0)), constant_values=-1.0)
    r_p = jnp.pad(rels.reshape(N, DEG * EDIM), ((0, pad), (0, 0)))
    nb_p = jnp.pad(neighbors.astype(jnp.int32), ((0, pad), (0, 0)))
    Wv = Wkqv[:, 2 * HID:]
    Wtv = Wtime[:, 2 * HID:]
    Wts, Wtc = Wtv[0::2], Wtv[1::2]
    We = Wedge[:, 2 * HID:]
    W1a, W1b = Wl1[:HID], Wl1[HID:]
    bp2 = bp.reshape(1, HID)
    LREP = DEG * (TDIM // 2)
    Rm = jnp.repeat(jnp.eye(DEG, dtype=f32), TDIM // 2, axis=1)
    wt_t = jnp.tile(Wt.reshape(1, TDIM // 2), (1, DEG))
    bt_t = jnp.tile(bt.reshape(1, TDIM // 2), (1, DEG))
    SWs = jnp.tile(Wts, (DEG, 1))
    SWc = jnp.tile(Wtc, (DEG, 1))
    SWe = jnp.tile(We, (DEG, 1))
    g1, b1 = ln1_g.reshape(1, HID), ln1_b.reshape(1, HID)
    g2, b2 = ln2_g.reshape(1, HID), ln2_b.reshape(1, HID)
    bl1r, bl2r = bl1.reshape(1, HID), bl2.reshape(1, HID)
    bor = bout.reshape(1, OUT)

    idx2d, xn, v = pl.pallas_call(
        _pre1_body,
        grid=(GRID,),
        in_specs=[
            _full_spec((1, 1)), _full_spec((1, 1)),
            _row_spec(),
            pl.BlockSpec((BN // CN, CN * DEG), lambda i: (i, 0)),
            pl.BlockSpec((BN // CN, CN * DEG), lambda i: (i, 0)),
            _full_spec((HID, HID)), _full_spec((1, HID)),
            _full_spec((1, HID)), _full_spec((1, HID)),
            _full_spec((HID, HID)),
        ],
        out_specs=[pl.BlockSpec((BN // CN, CN * DEG), lambda i: (i, 0)),
                   _row_spec(), _row_spec()],
        out_shape=[
            jax.ShapeDtypeStruct((N_PAD // CN, CN * DEG), jnp.int32),
            jax.ShapeDtypeStruct((N_PAD, HID), f32),
            jax.ShapeDtypeStruct((N_PAD, HID), f32),
        ],
    )(st, et, x_p, t_p.reshape(N_PAD // CN, CN * DEG),
      nb_p.reshape(N_PAD // CN, CN * DEG), Wp, bp2, g1, b1, Wv)

    tbm = pl.pallas_call(
        _pre2_body,
        grid=(GRID,),
        in_specs=[
            _full_spec((1, 1)), _full_spec((1, 1)),
            pl.BlockSpec((BN, DEG), lambda i: (i, 0)),
            pl.BlockSpec((BN, DEG * EDIM), lambda i: (i, 0)),
            _full_spec((DEG, LREP)),
            _full_spec((1, LREP)), _full_spec((1, LREP)),
            _full_spec((LREP, HID)), _full_spec((LREP, HID)),
            _full_spec((LREP, HID)),
        ],
        out_specs=_row_spec(),
        out_shape=jax.ShapeDtypeStruct((N_PAD, HID), f32),
    )(st, et, t_p, r_p, Rm, wt_t, bt_t, SWs, SWc, SWe)

    didx = (jnp.arange(16, dtype=jnp.int32)[:, None, None] * (NBUF * CN)
            + jnp.arange(NBUF, dtype=jnp.int32)[None, :, None] * CN
            + jnp.repeat(jnp.arange(CN, dtype=jnp.int32), DEG)[None, None, :]
            ).reshape(16 * NBUF * CN * DEG)

    mlp_lnv = pl.pallas_call(
        _mlp_lnv_body,
        grid=(GRID,),
        in_specs=[_row_spec(), _row_spec(), _row_spec(),
                  _full_spec((1, HID)), _full_spec((1, HID)),
                  _full_spec((HID, HID)), _full_spec((HID, HID)),
                  _full_spec((1, HID)), _full_spec((HID, HID)),
                  _full_spec((1, HID)),
                  _full_spec((1, HID)), _full_spec((1, HID)),
                  _full_spec((HID, HID))],
        out_specs=[_row_spec(), _row_spec()],
        out_shape=[jax.ShapeDtypeStruct((N_PAD, HID), f32),
                   jax.ShapeDtypeStruct((N_PAD, HID), f32)],
    )

    mlp_final = pl.pallas_call(
        _mlp_final_body,
        grid=(GRID,),
        in_specs=[_row_spec(), _row_spec(), _row_spec(),
                  _full_spec((1, HID)), _full_spec((1, HID)),
                  _full_spec((HID, HID)), _full_spec((HID, HID)),
                  _full_spec((1, HID)), _full_spec((HID, HID)),
                  _full_spec((1, HID)), _full_spec((HID, OUT)),
                  _full_spec((1, OUT))],
        out_specs=pl.BlockSpec((BN, OUT), lambda i: (i, 0)),
        out_shape=jax.ShapeDtypeStruct((N, OUT), f32),
    )

    agg = _make_gather_sum()(v, idx2d, didx)
    xn, v = mlp_lnv(xn, agg, tbm, g2, b2, W1a, W1b, bl1r, Wl2, bl2r,
                    g1, b1, Wv)
    agg = _make_gather_sum()(v, idx2d, didx)
    return mlp_final(xn, agg, tbm, g2, b2, W1a, W1b, bl1r, Wl2, bl2r, Wout,
                     bor)


# speedup vs baseline: 1.2366x; 1.0294x over previous
import functools

import jax
import jax.numpy as jnp
from jax import lax
from jax.experimental import pallas as pl
from jax.experimental.pallas import tpu as pltpu
from jax.experimental.pallas import tpu_sc as plsc

N = 10000
DEG = 16
HID = 128
TDIM = 32
EDIM = 16
OUT = 128
T_NORM = (1.0 / (TDIM // 2)) ** 0.5
EPS = 1e-5

NW = 32
N_PAD = 10240
NPW = N_PAD // NW
CN = 8
BN = 256
GRID = N_PAD // BN


def _ln(h, g, b):
    m = jnp.mean(h, axis=-1, keepdims=True)
    v = jnp.mean((h - m) ** 2, axis=-1, keepdims=True)
    return (h - m) * jax.lax.rsqrt(v + EPS) * g + b


def _pre1_body(st_ref, et_ref, x_ref, t_ref, nb_ref, Wp_ref, bp_ref,
               g_ref, b_ref, Wv_ref, idx_ref, xn_ref, v_ref):
    st = st_ref[0, 0]
    et = et_ref[0, 0]
    t = t_ref[...]
    mask = (t >= st) & (t < et)
    h0 = jnp.maximum(
        jnp.dot(x_ref[...], Wp_ref[...], preferred_element_type=jnp.float32)
        + bp_ref[...], 0.0)
    xn = _ln(h0, g_ref[...], b_ref[...])
    xn_ref[...] = xn
    v = jnp.dot(xn, Wv_ref[...], preferred_element_type=jnp.float32)
    row = (pl.program_id(0) * BN
           + lax.broadcasted_iota(jnp.int32, (BN, 1), 0))
    v_ref[...] = jnp.where(row < N, v, 0.0)
    idx_ref[...] = jnp.where(mask, nb_ref[...], N)


def _pre2_body(st_ref, et_ref, t_ref, r_ref, R_ref, wt_ref, bt_ref,
               SWs_ref, SWc_ref, SWe_ref, tbm_ref):
    st = st_ref[0, 0]
    et = et_ref[0, 0]
    t = t_ref[...]
    mask = (t >= st) & (t < et)
    maskf = mask.astype(jnp.float32)
    anymask = jnp.max(maskf, axis=1, keepdims=True)
    tmax = jnp.maximum(st, jnp.max(jnp.where(mask, t, -jnp.inf), axis=1,
                                   keepdims=True))
    Rm = R_ref[...]
    t_rep = jnp.dot(t, Rm, preferred_element_type=jnp.float32)
    m_rep = jnp.dot(maskf, Rm, preferred_element_type=jnp.float32)
    hh = (tmax - t_rep) * wt_ref[...] + bt_ref[...]
    tb = (jnp.dot(m_rep * jnp.sin(hh), SWs_ref[...],
                  preferred_element_type=jnp.float32)
          + jnp.dot(m_rep * jnp.cos(hh), SWc_ref[...],
                    preferred_element_type=jnp.float32)) * T_NORM
    tb = tb + jnp.dot(m_rep * r_ref[...], SWe_ref[...],
                      preferred_element_type=jnp.float32)
    tbm_ref[...] = anymask * tb


def _mlp_lnv_body(xn_ref, agg_ref, tbm_ref, g2_ref, b2_ref, W1a_ref, W1b_ref,
                  bl1_ref, W2_ref, bl2_ref, g1_ref, b1_ref, Wv_ref,
                  xn2_ref, v_ref):
    xn = xn_ref[...]
    h2 = agg_ref[...] + tbm_ref[...] + xn
    hn = _ln(h2, g2_ref[...], b2_ref[...])
    z = jnp.maximum(
        jnp.dot(xn, W1a_ref[...], preferred_element_type=jnp.float32)
        + jnp.dot(hn, W1b_ref[...], preferred_element_type=jnp.float32)
        + bl1_ref[...], 0.0)
    z = jnp.dot(z, W2_ref[...], preferred_element_type=jnp.float32) + bl2_ref[...]
    h = z + h2
    xn2 = _ln(h, g1_ref[...], b1_ref[...])
    xn2_ref[...] = xn2
    v = jnp.dot(xn2, Wv_ref[...], preferred_element_type=jnp.float32)
    row = (pl.program_id(0) * BN
           + lax.broadcasted_iota(jnp.int32, (BN, 1), 0))
    v_ref[...] = jnp.where(row < N, v, 0.0)


NCH = NPW // CN
NBUF = 4


@functools.cache
def _make_gather_sum():
    mesh = plsc.VectorSubcoreMesh(core_axis_name="c", subcore_axis_name="s")

    @functools.partial(
        pl.kernel,
        out_type=jax.ShapeDtypeStruct((N_PAD, HID), jnp.float32),
        mesh=mesh,
        scratch_types=[
            pltpu.VMEM((NCH, CN * DEG), jnp.int32),
            pltpu.VMEM((CN * DEG,), jnp.int32),
            pltpu.VMEM((CN * DEG,), jnp.int32),
            pltpu.VMEM((CN * DEG,), jnp.int32),
            pltpu.VMEM((CN * DEG,), jnp.int32),
            pltpu.VMEM((CN, HID), jnp.float32),
            pltpu.VMEM((NBUF, CN * DEG, HID), jnp.float32),
            pltpu.VMEM_SHARED((16 * NBUF * CN, HID), jnp.float32),
            pltpu.SemaphoreType.DMA,
            pltpu.SemaphoreType.DMA,
            pltpu.SemaphoreType.DMA,
            pltpu.SemaphoreType.DMA,
            pltpu.SemaphoreType.DMA,
            pltpu.SemaphoreType.DMA,
            pltpu.SemaphoreType.DMA,
            pltpu.SemaphoreType.DMA,
        ],
    )
    def _gather_sum(v_hbm, idx_hbm, didx_hbm, out_hbm, idx_s, d0, d1, d2, d3,
                    zero_s, rows_s, acc_sh, g0, g1, g2, g3, o0, o1, o2, o3):
        didx_s = (d0, d1, d2, d3)
        gsem = (g0, g1, g2, g3)
        osem = (o0, o1, o2, o3)
        sid = lax.axis_index("s")
        wid = sid * 2 + lax.axis_index("c")
        base = wid * NPW
        pltpu.sync_copy(idx_hbm.at[pl.ds(wid * NCH, NCH)], idx_s)
        for b in range(NBUF):
            pltpu.sync_copy(
                didx_hbm.at[pl.ds((sid * NBUF + b) * (CN * DEG), CN * DEG)],
                didx_s[b])
        for c in range(CN):
            for j in range(HID // 16):
                zero_s[c, pl.ds(j * 16, 16)] = jnp.zeros((16,), jnp.float32)

        def acc_rows(b):
            return acc_sh.at[pl.ds(sid * (NBUF * CN) + b * CN, CN)]

        def issue_gather(ci, b):
            return pltpu.async_copy(v_hbm.at[idx_s.at[ci]], rows_s.at[b],
                                    gsem[b])

        for b in range(NBUF):
            issue_gather(b, b)

        def group(g, carry):
            for b in range(NBUF):
                ci = g * NBUF + b
                node0 = base + ci * CN
                pltpu.make_async_copy(v_hbm.at[idx_s.at[ci]], rows_s.at[b],
                                      gsem[b]).wait()

                @pl.when(g > 0)
                def _wait_out():
                    pltpu.make_async_copy(
                        acc_rows(b), out_hbm.at[pl.ds(node0, CN)],
                        osem[b]).wait()

                pltpu.sync_copy(zero_s, acc_rows(b))
                pltpu.sync_copy(rows_s.at[b], acc_sh.at[didx_s[b]],
                                add=True)
                nc = ci + NBUF

                @pl.when(nc < NCH)
                def _next():
                    issue_gather(nc, b)

            plsc.subcore_barrier()
            for b in range(NBUF):
                node0 = base + (g * NBUF + b) * CN
                pltpu.async_copy(acc_rows(b), out_hbm.at[pl.ds(node0, CN)],
                                 osem[b])
            return carry

        lax.fori_loop(0, NCH // NBUF, group, 0)
        for b in range(NBUF):
            node0 = base + (NCH - NBUF + b) * CN
            pltpu.make_async_copy(acc_rows(b), out_hbm.at[pl.ds(node0, CN)],
                                  osem[b]).wait()

    return _gather_sum


def _mlp_final_body(xn_ref, agg_ref, tbm_ref, g_ref, b_ref, W1a_ref, W1b_ref,
                    bl1_ref, W2_ref, bl2_ref, Wo_ref, bo_ref, o_ref):
    xn = xn_ref[...]
    h2 = agg_ref[...] + tbm_ref[...] + xn
    hn = _ln(h2, g_ref[...], b_ref[...])
    z = jnp.maximum(
        jnp.dot(xn, W1a_ref[...], preferred_element_type=jnp.float32)
        + jnp.dot(hn, W1b_ref[...], preferred_element_type=jnp.float32)
        + bl1_ref[...], 0.0)
    z = jnp.dot(z, W2_ref[...], preferred_element_type=jnp.float32) + bl2_ref[...]
    hnxt = z + h2
    o_ref[...] = (jnp.dot(hnxt, Wo_ref[...], preferred_element_type=jnp.float32)
                  + bo_ref[...])


def _row_spec():
    return pl.BlockSpec((BN, HID), lambda i: (i, 0))


def _full_spec(shape):
    return pl.BlockSpec(shape, lambda i: tuple(0 for _ in shape))


def kernel(x, neighbors, times, rels, start_t, end_t, Wp, bp, ln1_g, ln1_b,
           Wkqv, Wt, bt, Wtime, Wedge, ln2_g, ln2_b, Wl1, bl1, Wl2, bl2,
           Wout, bout):
    f32 = jnp.float32
    st = jnp.asarray(start_t, f32).reshape(1, 1)
    et = jnp.asarray(end_t, f32).reshape(1, 1)

    pad = N_PAD - N
    t_p = jnp.pad(times[:, :, 0], ((0, pad), (0, 0)), constant_values=-1.0)
    r_p = jnp.pad(rels.reshape(N, DEG * EDIM), ((0, pad), (0, 0)))
    nb_p = jnp.pad(neighbors.astype(jnp.int32), ((0, pad), (0, 0)))
    Wv = Wkqv[:, 2 * HID:]
    Wtv = Wtime[:, 2 * HID:]
    Wts, Wtc = Wtv[0::2], Wtv[1::2]
    We = Wedge[:, 2 * HID:]
    W1a, W1b = Wl1[:HID], Wl1[HID:]
    bp2 = bp.reshape(1, HID)
    LREP = DEG * (TDIM // 2)
    Rm = jnp.repeat(jnp.eye(DEG, dtype=f32), TDIM // 2, axis=1)
    wt_t = jnp.tile(Wt.reshape(1, TDIM // 2), (1, DEG))
    bt_t = jnp.tile(bt.reshape(1, TDIM // 2), (1, DEG))
    SWs = jnp.tile(Wts, (DEG, 1))
    SWc = jnp.tile(Wtc, (DEG, 1))
    SWe = jnp.tile(We, (DEG, 1))
    g1, b1 = ln1_g.reshape(1, HID), ln1_b.reshape(1, HID)
    g2, b2 = ln2_g.reshape(1, HID), ln2_b.reshape(1, HID)
    bl1r, bl2r = bl1.reshape(1, HID), bl2.reshape(1, HID)
    bor = bout.reshape(1, OUT)

    idx2d, xn, v = pl.pallas_call(
        _pre1_body,
        grid=(GRID,),
        in_specs=[
            _full_spec((1, 1)), _full_spec((1, 1)),
            _row_spec(),
            pl.BlockSpec((BN // CN, CN * DEG), lambda i: (i, 0)),
            pl.BlockSpec((BN // CN, CN * DEG), lambda i: (i, 0)),
            _full_spec((HID, HID)), _full_spec((1, HID)),
            _full_spec((1, HID)), _full_spec((1, HID)),
            _full_spec((HID, HID)),
        ],
        out_specs=[pl.BlockSpec((BN // CN, CN * DEG), lambda i: (i, 0)),
                   _row_spec(), _row_spec()],
        out_shape=[
            jax.ShapeDtypeStruct((N_PAD // CN, CN * DEG), jnp.int32),
            jax.ShapeDtypeStruct((N_PAD, HID), f32),
            jax.ShapeDtypeStruct((N_PAD, HID), f32),
        ],
    )(st, et, x, t_p.reshape(N_PAD // CN, CN * DEG),
      nb_p.reshape(N_PAD // CN, CN * DEG), Wp, bp2, g1, b1, Wv)

    tbm = pl.pallas_call(
        _pre2_body,
        grid=(GRID,),
        in_specs=[
            _full_spec((1, 1)), _full_spec((1, 1)),
            pl.BlockSpec((BN, DEG), lambda i: (i, 0)),
            pl.BlockSpec((BN, DEG * EDIM), lambda i: (i, 0)),
            _full_spec((DEG, LREP)),
            _full_spec((1, LREP)), _full_spec((1, LREP)),
            _full_spec((LREP, HID)), _full_spec((LREP, HID)),
            _full_spec((LREP, HID)),
        ],
        out_specs=_row_spec(),
        out_shape=jax.ShapeDtypeStruct((N_PAD, HID), f32),
    )(st, et, t_p, r_p, Rm, wt_t, bt_t, SWs, SWc, SWe)

    didx = (jnp.arange(16, dtype=jnp.int32)[:, None, None] * (NBUF * CN)
            + jnp.arange(NBUF, dtype=jnp.int32)[None, :, None] * CN
            + jnp.repeat(jnp.arange(CN, dtype=jnp.int32), DEG)[None, None, :]
            ).reshape(16 * NBUF * CN * DEG)

    mlp_lnv = pl.pallas_call(
        _mlp_lnv_body,
        grid=(GRID,),
        in_specs=[_row_spec(), _row_spec(), _row_spec(),
                  _full_spec((1, HID)), _full_spec((1, HID)),
                  _full_spec((HID, HID)), _full_spec((HID, HID)),
                  _full_spec((1, HID)), _full_spec((HID, HID)),
                  _full_spec((1, HID)),
                  _full_spec((1, HID)), _full_spec((1, HID)),
                  _full_spec((HID, HID))],
        out_specs=[_row_spec(), _row_spec()],
        out_shape=[jax.ShapeDtypeStruct((N_PAD, HID), f32),
                   jax.ShapeDtypeStruct((N_PAD, HID), f32)],
    )

    mlp_final = pl.pallas_call(
        _mlp_final_body,
        grid=(GRID,),
        in_specs=[_row_spec(), _row_spec(), _row_spec(),
                  _full_spec((1, HID)), _full_spec((1, HID)),
                  _full_spec((HID, HID)), _full_spec((HID, HID)),
                  _full_spec((1, HID)), _full_spec((HID, HID)),
                  _full_spec((1, HID)), _full_spec((HID, OUT)),
                  _full_spec((1, OUT))],
        out_specs=pl.BlockSpec((BN, OUT), lambda i: (i, 0)),
        out_shape=jax.ShapeDtypeStruct((N, OUT), f32),
    )

    agg = _make_gather_sum()(v, idx2d, didx)
    xn, v = mlp_lnv(xn, agg, tbm, g2, b2, W1a, W1b, bl1r, Wl2, bl2r,
                    g1, b1, Wv)
    agg = _make_gather_sum()(v, idx2d, didx)
    return mlp_final(xn, agg, tbm, g2, b2, W1a, W1b, bl1r, Wl2, bl2r, Wout,
                     bor)
